# Initial kernel scaffold; baseline (speedup 1.0000x reference)
import jax
import jax.numpy as jnp
from jax.experimental import pallas as pl


def kernel(x, edge_index, enc_W, enc_b, Wl, bl, Wr, gamma, beta, dec_W, dec_b):
    raise NotImplementedError("write your pallas kernel here")


# speedup vs baseline: 8.7173x; 8.7173x over previous
import functools

import jax
import jax.numpy as jnp
from jax import lax
from jax.experimental import pallas as pl
from jax.experimental.pallas import tpu as pltpu
from jax.experimental.pallas import tpu_sc as plsc

N_NODES = 10000
N_EDGES = 320000
D_IN = 128
H = 64
D_OUT = 4

NC = 2
NS = 16
NW = NC * NS
EPT = N_EDGES // NW
CHUNK = 80
NCH = EPT // CHUNK
RPT = N_NODES // NS


def _sc_agg(with_counts: bool):
  mesh = plsc.VectorSubcoreMesh(core_axis_name="c", subcore_axis_name="s")
  out_type = [jax.ShapeDtypeStruct((NC, N_NODES, H), jnp.float32)]
  if with_counts:
    out_type.append(jax.ShapeDtypeStruct((NW, N_NODES), jnp.float32))

  scratch = [
      pltpu.VMEM((NCH, CHUNK), jnp.int32),
      pltpu.VMEM((NCH, CHUNK), jnp.int32),
      pltpu.VMEM((CHUNK, H), jnp.float32),
      pltpu.VMEM((RPT, H), jnp.float32),
      pltpu.VMEM((N_NODES,), jnp.float32),
      pltpu.VMEM_SHARED((N_NODES, H), jnp.float32),
      pltpu.SemaphoreType.DMA,
  ]

  def body(*refs):
    if with_counts:
      (h_hbm, src_hbm, dst_hbm, agg_out, cnt_out,
       src_v, dst_v, rows_v, bounce_v, cnt_v, agg_sh, sem) = refs
    else:
      (h_hbm, src_hbm, dst_hbm, agg_out,
       src_v, dst_v, rows_v, bounce_v, cnt_v, agg_sh, sem) = refs

    c = lax.axis_index("c")
    s = lax.axis_index("s")
    wid = c * NS + s

    zeros16 = jnp.zeros((16,), jnp.float32)

    def zero_row(i, carry):
      for k in range(H // 16):
        bounce_v[i, pl.ds(k * 16, 16)] = zeros16
      return carry

    lax.fori_loop(0, RPT, zero_row, 0)
    pltpu.sync_copy(bounce_v, agg_sh.at[pl.ds(s * RPT, RPT)])

    if with_counts:
      def zero_cnt(i, carry):
        cnt_v[pl.ds(i * 16, 16)] = zeros16
        return carry
      lax.fori_loop(0, N_NODES // 16, zero_cnt, 0)

    pltpu.sync_copy(src_hbm.at[wid], src_v)
    pltpu.sync_copy(dst_hbm.at[wid], dst_v)
    plsc.subcore_barrier()

    ones16 = jnp.ones((16,), jnp.float32)

    def step(j, carry):
      pltpu.async_copy(h_hbm.at[src_v.at[j]], rows_v, sem).wait()
      pltpu.sync_copy(rows_v, agg_sh.at[dst_v.at[j]], add=True)
      if with_counts:
        for k in range(CHUNK // 16):
          idx = dst_v[j, pl.ds(k * 16, 16)]
          plsc.addupdate_scatter(cnt_v, [idx], ones16)
      return carry

    lax.fori_loop(0, NCH, step, 0)
    plsc.subcore_barrier()

    pltpu.sync_copy(agg_sh.at[pl.ds(s * RPT, RPT)], bounce_v)
    pltpu.sync_copy(bounce_v, agg_out.at[c, pl.ds(s * RPT, RPT)])
    if with_counts:
      pltpu.sync_copy(cnt_v, cnt_out.at[wid])

  return pl.kernel(
      body, out_type=out_type, mesh=mesh, scratch_types=scratch,
      compiler_params=pltpu.CompilerParams(
          use_tc_tiling_on_sc=False, needs_layout_passes=False))


_sc_agg_first = _sc_agg(with_counts=True)
_sc_agg_rest = _sc_agg(with_counts=False)


def _enc_body(x_ref, w_ref, b_ref, wl_ref, z_ref, h_ref):
  z = jnp.dot(x_ref[...], w_ref[...], preferred_element_type=jnp.float32)
  z = jnp.maximum(z + b_ref[...], 0.0)
  z_ref[...] = z
  h_ref[...] = jnp.dot(z, wl_ref[...], preferred_element_type=jnp.float32)


_enc = pl.pallas_call(
    _enc_body,
    out_shape=[
        jax.ShapeDtypeStruct((N_NODES, H), jnp.float32),
        jax.ShapeDtypeStruct((N_NODES, H), jnp.float32),
    ],
)


def _mid_body(agg_ref, cntp_ref, z_ref, wr_ref, bl_ref, g_ref, b_ref,
              wl_ref, u_ref, h_ref):
  cnt = jnp.maximum(jnp.sum(cntp_ref[...], axis=0), 1.0)
  inv = (1.0 / cnt)[:, None]
  t = (agg_ref[0] + agg_ref[1]) * inv + bl_ref[...]
  t = t + jnp.dot(z_ref[...], wr_ref[...], preferred_element_type=jnp.float32)
  mu = jnp.mean(t, axis=0, keepdims=True)
  var = jnp.mean((t - mu) ** 2, axis=0, keepdims=True)
  u = g_ref[...] * (t - mu) * lax.rsqrt(var + 1e-5) + b_ref[...]
  u = jnp.maximum(u, 0.0)
  u_ref[...] = u
  h_ref[...] = jnp.dot(u, wl_ref[...], preferred_element_type=jnp.float32)


_mid = pl.pallas_call(
    _mid_body,
    out_shape=[
        jax.ShapeDtypeStruct((N_NODES, H), jnp.float32),
        jax.ShapeDtypeStruct((N_NODES, H), jnp.float32),
    ],
)


def _fin_body(agg_ref, cntp_ref, z_ref, wr_ref, bl_ref, dw_ref, db_ref,
              out_ref):
  cnt = jnp.maximum(jnp.sum(cntp_ref[...], axis=0), 1.0)
  inv = (1.0 / cnt)[:, None]
  t = (agg_ref[0] + agg_ref[1]) * inv + bl_ref[...]
  t = t + jnp.dot(z_ref[...], wr_ref[...], preferred_element_type=jnp.float32)
  out_ref[...] = (
      jnp.dot(t, dw_ref[...], preferred_element_type=jnp.float32)
      + db_ref[...])


_fin = pl.pallas_call(
    _fin_body,
    out_shape=jax.ShapeDtypeStruct((N_NODES, D_OUT), jnp.float32),
)


def kernel(x, edge_index, enc_W, enc_b, Wl, bl, Wr, gamma, beta, dec_W, dec_b):
  src = edge_index[0].reshape(NW, NCH, CHUNK)
  dst = edge_index[1].reshape(NW, NCH, CHUNK)

  z, h = _enc(x, enc_W, enc_b.reshape(1, H), Wl[0])
  agg, cntp = _sc_agg_first(h, src, dst)
  for n in range(3):
    z, h = _mid(agg, cntp, z, Wr[n], bl[n].reshape(1, H),
                gamma[n].reshape(1, H), beta[n].reshape(1, H), Wl[n + 1])
    (agg,) = _sc_agg_rest(h, src, dst)
  return _fin(agg, cntp, z, Wr[3], bl[3].reshape(1, H), dec_W,
              dec_b.reshape(1, D_OUT))
